# Initial kernel scaffold; baseline (speedup 1.0000x reference)
import jax
import jax.numpy as jnp
from jax.experimental import pallas as pl


def kernel(idx, targets, table):
    raise NotImplementedError("write your pallas kernel here")


# speedup vs baseline: 1.2554x; 1.2554x over previous
import functools

import jax
import jax.numpy as jnp
from jax import lax
from jax.experimental import pallas as pl
from jax.experimental.pallas import tpu as pltpu
from jax.experimental.pallas import tpu_sc as plsc

V = 1000
B = 32
T = 1024
NTOK = B * T

_info = plsc.get_sparse_core_info()
NC = _info.num_cores
NS = _info.num_subcores
L = _info.num_lanes
NW = NC * NS
BPW = NTOK // NW
CH = 64


def _row_lse_body(tab_ref, out_ref):
    x = tab_ref[...]
    m = jnp.max(x, axis=1)
    s = jnp.sum(jnp.exp(x - m[:, None]), axis=1)
    out_ref[...] = jnp.log(s) + m


def _row_lse(table):
    return pl.pallas_call(
        _row_lse_body,
        out_shape=jax.ShapeDtypeStruct((V,), jnp.float32),
    )(table)


@functools.partial(
    pl.kernel,
    mesh=plsc.VectorSubcoreMesh(core_axis_name="c", subcore_axis_name="s"),
    compiler_params=pltpu.CompilerParams(
        use_tc_tiling_on_sc=False, needs_layout_passes=False),
    out_type=[
        jax.ShapeDtypeStruct((NTOK, V), jnp.float32),
        jax.ShapeDtypeStruct((NW, L), jnp.float32),
    ],
    scratch_types=[
        pltpu.VMEM((BPW,), jnp.int32),
        pltpu.VMEM((BPW,), jnp.int32),
        pltpu.VMEM((BPW,), jnp.int32),
        pltpu.VMEM((CH, V), jnp.float32),
        pltpu.VMEM((CH,), jnp.float32),
        pltpu.VMEM((V,), jnp.float32),
        pltpu.VMEM((L,), jnp.float32),
        pltpu.SemaphoreType.DMA,
        pltpu.SemaphoreType.DMA,
    ],
)
def _sc_gather(idx_hbm, tgt_hbm, lse_hbm, table_hbm, tflat_hbm,
               out_hbm, part_hbm,
               idx_v, tgt_v, fidx_v, rows_v, pick_v, lse_v, acc_v,
               sem, sem2):
    wid = lax.axis_index("s") * NC + lax.axis_index("c")
    wbase = wid * BPW
    pltpu.sync_copy(idx_hbm.at[pl.ds(wbase, BPW)], idx_v)
    pltpu.sync_copy(tgt_hbm.at[pl.ds(wbase, BPW)], tgt_v)
    pltpu.sync_copy(lse_hbm, lse_v)
    acc_v[...] = jnp.full((L,), 0.0, jnp.float32)

    def fidx_body(g, carry):
        sl = pl.ds(g * L, L)
        fidx_v[sl] = idx_v[sl] * V + tgt_v[sl]
        return carry

    lax.fori_loop(0, BPW // L, fidx_body, 0)

    def chunk_body(c, carry):
        base = c * CH
        cp_rows = pltpu.async_copy(
            table_hbm.at[idx_v.at[pl.ds(base, CH)]], rows_v, sem)
        cp_pick = pltpu.async_copy(
            tflat_hbm.at[fidx_v.at[pl.ds(base, CH)]], pick_v, sem2)
        cp_rows.wait()
        pltpu.sync_copy(rows_v, out_hbm.at[pl.ds(wbase + base, CH)])
        cp_pick.wait()
        for g in range(CH // L):
            sl = pl.ds(base + g * L, L)
            lse16 = plsc.load_gather(lse_v, [idx_v[sl]])
            acc_v[...] = acc_v[...] + (lse16 - pick_v[pl.ds(g * L, L)])
        return carry

    lax.fori_loop(0, BPW // CH, chunk_body, 0)
    pltpu.sync_copy(acc_v, part_hbm.at[wid])


def _finish_body(p_ref, out_ref):
    out_ref[...] = (jnp.sum(p_ref[...]) / jnp.float32(NTOK)).reshape(1, 1)


def _finish(partials):
    out = pl.pallas_call(
        _finish_body,
        out_shape=jax.ShapeDtypeStruct((1, 1), jnp.float32),
    )(partials)
    return out[0, 0]


def kernel(idx, targets, table):
    idx_f = idx.reshape(NTOK).astype(jnp.int32)
    tgt_f = targets.reshape(NTOK).astype(jnp.int32)
    table = table.astype(jnp.float32)
    row_lse = _row_lse(table)
    tflat = jnp.concatenate([table.reshape(V * V), jnp.zeros(8, jnp.float32)])
    logits_flat, partials = _sc_gather(idx_f, tgt_f, row_lse, table, tflat)
    loss = _finish(partials)
    return (logits_flat.reshape(B, T, V), loss)


# speedup vs baseline: 1.2716x; 1.0129x over previous
import functools

import jax
import jax.numpy as jnp
from jax import lax
from jax.experimental import pallas as pl
from jax.experimental.pallas import tpu as pltpu
from jax.experimental.pallas import tpu_sc as plsc

V = 1000
B = 32
T = 1024
NTOK = B * T

_info = plsc.get_sparse_core_info()
NC = _info.num_cores
NS = _info.num_subcores
L = _info.num_lanes
NW = NC * NS
BPW = NTOK // NW
CH = 32
NCH = BPW // CH
PK = 128


def _row_lse_body(tab_ref, out_ref):
    x = tab_ref[...]
    m = jnp.max(x, axis=1)
    s = jnp.sum(jnp.exp(x - m[:, None]), axis=1)
    out_ref[...] = jnp.log(s) + m


def _row_lse(table):
    return pl.pallas_call(
        _row_lse_body,
        out_shape=jax.ShapeDtypeStruct((V,), jnp.float32),
    )(table)


@functools.partial(
    pl.kernel,
    mesh=plsc.VectorSubcoreMesh(core_axis_name="c", subcore_axis_name="s"),
    compiler_params=pltpu.CompilerParams(
        use_tc_tiling_on_sc=False, needs_layout_passes=False),
    out_type=[
        jax.ShapeDtypeStruct((NTOK, V), jnp.float32),
        jax.ShapeDtypeStruct((NW, L), jnp.float32),
    ],
    scratch_types=[
        pltpu.VMEM((BPW,), jnp.int32),
        pltpu.VMEM((BPW,), jnp.int32),
        pltpu.VMEM((BPW,), jnp.int32),
        pltpu.VMEM((CH, V), jnp.float32),
        pltpu.VMEM((CH, V), jnp.float32),
        pltpu.VMEM((BPW,), jnp.float32),
        pltpu.VMEM((V,), jnp.float32),
        pltpu.VMEM((L,), jnp.float32),
        pltpu.SemaphoreType.DMA,
        pltpu.SemaphoreType.DMA,
        pltpu.SemaphoreType.DMA,
        pltpu.SemaphoreType.DMA,
        pltpu.SemaphoreType.DMA,
    ],
)
def _sc_gather(idx_hbm, tgt_hbm, lse_hbm, table_hbm, tflat_hbm,
               out_hbm, part_hbm,
               idx_v, tgt_v, fidx_v, rows_a, rows_b, pick_v, lse_v, acc_v,
               gsem_a, gsem_b, wsem_a, wsem_b, psem):
    wid = lax.axis_index("s") * NC + lax.axis_index("c")
    wbase = wid * BPW
    pltpu.sync_copy(idx_hbm.at[pl.ds(wbase, BPW)], idx_v)
    pltpu.sync_copy(tgt_hbm.at[pl.ds(wbase, BPW)], tgt_v)
    pltpu.sync_copy(lse_hbm, lse_v)

    def fidx_body(g, carry):
        sl = pl.ds(g * L, L)
        fidx_v[sl] = idx_v[sl] * V + tgt_v[sl]
        return carry

    lax.fori_loop(0, BPW // L, fidx_body, 0)

    pick_cps = [
        pltpu.async_copy(tflat_hbm.at[fidx_v.at[pl.ds(j * PK, PK)]],
                         pick_v.at[pl.ds(j * PK, PK)], psem)
        for j in range(BPW // PK)
    ]

    def gather(c, buf, sem):
        pltpu.async_copy(table_hbm.at[idx_v.at[pl.ds(c * CH, CH)]], buf, sem)

    def writeout(c, buf, sem):
        pltpu.async_copy(buf, out_hbm.at[pl.ds(wbase + c * CH, CH)], sem)

    def gwait(buf, sem):
        pltpu.make_async_copy(table_hbm.at[pl.ds(0, CH)], buf, sem).wait()

    def wwait(buf, sem):
        pltpu.make_async_copy(buf, out_hbm.at[pl.ds(0, CH)], sem).wait()

    gather(0, rows_a, gsem_a)
    npairs = NCH // 2

    def pair_body(p, carry):
        c0 = p * 2
        gwait(rows_a, gsem_a)
        gather(c0 + 1, rows_b, gsem_b)
        writeout(c0, rows_a, wsem_a)
        gwait(rows_b, gsem_b)
        wwait(rows_a, wsem_a)

        @pl.when(p < npairs - 1)
        def _():
            gather(c0 + 2, rows_a, gsem_a)

        writeout(c0 + 1, rows_b, wsem_b)
        wwait(rows_b, wsem_b)
        return carry

    lax.fori_loop(0, npairs, pair_body, 0)

    for cp in pick_cps:
        cp.wait()
    acc_v[...] = jnp.full((L,), 0.0, jnp.float32)

    def loss_body(g, carry):
        sl = pl.ds(g * L, L)
        lse16 = plsc.load_gather(lse_v, [idx_v[sl]])
        acc_v[...] = acc_v[...] + (lse16 - pick_v[sl])
        return carry

    lax.fori_loop(0, BPW // L, loss_body, 0)
    pltpu.sync_copy(acc_v, part_hbm.at[wid])


def _finish_body(p_ref, out_ref):
    out_ref[...] = (jnp.sum(p_ref[...]) / jnp.float32(NTOK)).reshape(1, 1)


def _finish(partials):
    out = pl.pallas_call(
        _finish_body,
        out_shape=jax.ShapeDtypeStruct((1, 1), jnp.float32),
    )(partials)
    return out[0, 0]


def kernel(idx, targets, table):
    idx_f = idx.reshape(NTOK).astype(jnp.int32)
    tgt_f = targets.reshape(NTOK).astype(jnp.int32)
    table = table.astype(jnp.float32)
    row_lse = _row_lse(table)
    tflat = jnp.concatenate([table.reshape(V * V), jnp.zeros(8, jnp.float32)])
    logits_flat, partials = _sc_gather(idx_f, tgt_f, row_lse, table, tflat)
    loss = _finish(partials)
    return (logits_flat.reshape(B, T, V), loss)


# speedup vs baseline: 2.0753x; 1.6321x over previous
import functools

import jax
import jax.numpy as jnp
from jax import lax
from jax.experimental import pallas as pl
from jax.experimental.pallas import tpu as pltpu
from jax.experimental.pallas import tpu_sc as plsc

V = 1000
VP = 1024
B = 32
T = 1024
NTOK = B * T

_info = plsc.get_sparse_core_info()
NC = _info.num_cores
NS = _info.num_subcores
L = _info.num_lanes
NW = NC * NS
BPW = NTOK // NW
CH = 32
NCH = BPW // CH
PK = 128


def _row_lse_body(tab_ref, out_ref):
    x = tab_ref[...]
    m = jnp.max(x, axis=1)
    s = jnp.sum(jnp.exp(x - m[:, None]), axis=1)
    out_ref[...] = jnp.log(s) + m


def _row_lse(table):
    return pl.pallas_call(
        _row_lse_body,
        out_shape=jax.ShapeDtypeStruct((V,), jnp.float32),
    )(table)


@functools.partial(
    pl.kernel,
    mesh=plsc.VectorSubcoreMesh(core_axis_name="c", subcore_axis_name="s"),
    compiler_params=pltpu.CompilerParams(
        use_tc_tiling_on_sc=True, needs_layout_passes=False),
    out_type=[
        jax.ShapeDtypeStruct((NTOK, VP), jnp.float32),
        jax.ShapeDtypeStruct((NW, L), jnp.float32),
    ],
    scratch_types=[
        pltpu.VMEM((BPW,), jnp.int32),
        pltpu.VMEM((BPW,), jnp.int32),
        pltpu.VMEM((BPW,), jnp.int32),
        pltpu.VMEM((CH, VP), jnp.float32),
        pltpu.VMEM((CH, VP), jnp.float32),
        pltpu.VMEM((BPW,), jnp.float32),
        pltpu.VMEM((V,), jnp.float32),
        pltpu.VMEM((L,), jnp.float32),
        pltpu.SemaphoreType.DMA,
        pltpu.SemaphoreType.DMA,
        pltpu.SemaphoreType.DMA,
        pltpu.SemaphoreType.DMA,
        pltpu.SemaphoreType.DMA,
    ],
)
def _sc_gather(idx_hbm, tgt_hbm, lse_hbm, table_hbm, tflat_hbm,
               out_hbm, part_hbm,
               idx_v, tgt_v, fidx_v, rows_a, rows_b, pick_v, lse_v, acc_v,
               gsem_a, gsem_b, wsem_a, wsem_b, psem):
    wid = lax.axis_index("s") * NC + lax.axis_index("c")
    wbase = wid * BPW
    pltpu.sync_copy(idx_hbm.at[pl.ds(wbase, BPW)], idx_v)
    pltpu.sync_copy(tgt_hbm.at[pl.ds(wbase, BPW)], tgt_v)
    pltpu.sync_copy(lse_hbm, lse_v)

    def fidx_body(g, carry):
        sl = pl.ds(g * L, L)
        fidx_v[sl] = idx_v[sl] * VP + tgt_v[sl]
        return carry

    lax.fori_loop(0, BPW // L, fidx_body, 0)

    pick_cps = [
        pltpu.async_copy(tflat_hbm.at[fidx_v.at[pl.ds(j * PK, PK)]],
                         pick_v.at[pl.ds(j * PK, PK)], psem)
        for j in range(BPW // PK)
    ]

    def gather(c, buf, sem):
        pltpu.async_copy(table_hbm.at[idx_v.at[pl.ds(c * CH, CH)]], buf, sem)

    def writeout(c, buf, sem):
        pltpu.async_copy(buf, out_hbm.at[pl.ds(wbase + c * CH, CH)], sem)

    def gwait(buf, sem):
        pltpu.make_async_copy(table_hbm.at[pl.ds(0, CH)], buf, sem).wait()

    def wwait(buf, sem):
        pltpu.make_async_copy(buf, out_hbm.at[pl.ds(0, CH)], sem).wait()

    gather(0, rows_a, gsem_a)
    npairs = NCH // 2

    def pair_body(p, carry):
        c0 = p * 2
        gwait(rows_a, gsem_a)
        gather(c0 + 1, rows_b, gsem_b)
        writeout(c0, rows_a, wsem_a)
        gwait(rows_b, gsem_b)
        wwait(rows_a, wsem_a)

        @pl.when(p < npairs - 1)
        def _():
            gather(c0 + 2, rows_a, gsem_a)

        writeout(c0 + 1, rows_b, wsem_b)
        wwait(rows_b, wsem_b)
        return carry

    lax.fori_loop(0, npairs, pair_body, 0)

    for cp in pick_cps:
        cp.wait()
    acc_v[...] = jnp.full((L,), 0.0, jnp.float32)

    def loss_body(g, carry):
        sl = pl.ds(g * L, L)
        lse16 = plsc.load_gather(lse_v, [idx_v[sl]])
        acc_v[...] = acc_v[...] + (lse16 - pick_v[sl])
        return carry

    lax.fori_loop(0, BPW // L, loss_body, 0)
    pltpu.sync_copy(acc_v, part_hbm.at[wid])


def _finish_body(p_ref, out_ref):
    out_ref[...] = (jnp.sum(p_ref[...]) / jnp.float32(NTOK)).reshape(1, 1)


def _finish(partials):
    out = pl.pallas_call(
        _finish_body,
        out_shape=jax.ShapeDtypeStruct((1, 1), jnp.float32),
    )(partials)
    return out[0, 0]


def kernel(idx, targets, table):
    idx_f = idx.reshape(NTOK).astype(jnp.int32)
    tgt_f = targets.reshape(NTOK).astype(jnp.int32)
    table = table.astype(jnp.float32)
    table_p = jnp.pad(table, ((0, 0), (0, VP - V)))
    row_lse = _row_lse(table)
    tflat = jnp.concatenate(
        [table_p.reshape(V * VP), jnp.zeros(8, jnp.float32)])
    out2d, partials = _sc_gather(idx_f, tgt_f, row_lse, table_p, tflat)
    loss = _finish(partials)
    logits = out2d.reshape(B, T, VP)[:, :, :V]
    return (logits, loss)
